# Initial kernel scaffold; baseline (speedup 1.0000x reference)
import jax
import jax.numpy as jnp
from jax.experimental import pallas as pl


def kernel(x, adj, W_l0, b_l0, W_r0, W_l1, b_l1, W_r1, W_l2, b_l2, W_r2):
    raise NotImplementedError("write your pallas kernel here")


# speedup vs baseline: 824.5582x; 824.5582x over previous
import jax
import jax.numpy as jnp
from jax.experimental import pallas as pl


def _sage_fused(adj_ref, x_ref, ones_ref,
                wl0_ref, bl0_ref, wr0_ref,
                wl1_ref, bl1_ref, wr1_ref,
                wl2_ref, bl2_ref, wr2_ref,
                out_ref):
    adj = adj_ref[...]
    x = x_ref[...]

    deg = jax.lax.dot_general(
        adj, ones_ref[...], (((0,), (0,)), ((), ())),
        preferred_element_type=jnp.float32,
        precision=jax.lax.Precision.HIGHEST)
    dinv = 1.0 / jnp.maximum(deg, 1.0)

    def layer(h, wl_ref, bl_ref, wr_ref):
        agg = jax.lax.dot_general(
            adj, h, (((0,), (0,)), ((), ())),
            preferred_element_type=jnp.float32,
            precision=jax.lax.Precision.HIGHEST)
        mean = agg * dinv
        lin_l = jnp.dot(mean, wl_ref[...],
                        preferred_element_type=jnp.float32,
                        precision=jax.lax.Precision.HIGHEST)
        lin_r = jnp.dot(h, wr_ref[...],
                        preferred_element_type=jnp.float32,
                        precision=jax.lax.Precision.HIGHEST)
        return lin_l + lin_r + bl_ref[...]

    h = jnp.maximum(layer(x, wl0_ref, bl0_ref, wr0_ref), 0.0)
    h = jnp.maximum(layer(h, wl1_ref, bl1_ref, wr1_ref), 0.0)
    out_ref[...] = layer(h, wl2_ref, bl2_ref, wr2_ref)


def kernel(x, adj, W_l0, b_l0, W_r0, W_l1, b_l1, W_r1, W_l2, b_l2, W_r2):
    n, d_in = x.shape
    ones = jnp.ones((n, 1), dtype=jnp.float32)
    return pl.pallas_call(
        _sage_fused,
        out_shape=jax.ShapeDtypeStruct((n, W_l2.shape[1]), jnp.float32),
    )(adj, x, ones,
      W_l0, b_l0.reshape(1, -1), W_r0,
      W_l1, b_l1.reshape(1, -1), W_r1,
      W_l2, b_l2.reshape(1, -1), W_r2)


# speedup vs baseline: 1420.6029x; 1.7229x over previous
import jax
import jax.numpy as jnp
from jax.experimental import pallas as pl

_PREC = jax.lax.Precision.HIGHEST


def _sage_fused(adj_ref, gx_ref,
                wl0_ref, bl0_ref, wr0_ref,
                wl1_ref, bl1_ref, wr1_ref,
                wl2_ref, bl2_ref, wr2_ref,
                out_ref):
    adj = adj_ref[...]
    g = gx_ref[...]

    deg = jnp.sum(adj, axis=0, keepdims=True)
    dinv = 1.0 / jnp.maximum(deg, 1.0)

    def layer(gh, wlT_ref, blc_ref, wrT_ref):
        aggT = jnp.dot(gh, adj, preferred_element_type=jnp.float32,
                       precision=_PREC)
        meanT = aggT * dinv
        lin_l = jnp.dot(wlT_ref[...], meanT, preferred_element_type=jnp.float32,
                        precision=_PREC)
        lin_r = jnp.dot(wrT_ref[...], gh, preferred_element_type=jnp.float32,
                        precision=_PREC)
        return lin_l + lin_r + blc_ref[...]

    g = jnp.maximum(layer(g, wl0_ref, bl0_ref, wr0_ref), 0.0)
    g = jnp.maximum(layer(g, wl1_ref, bl1_ref, wr1_ref), 0.0)
    out_ref[...] = layer(g, wl2_ref, bl2_ref, wr2_ref)


def kernel(x, adj, W_l0, b_l0, W_r0, W_l1, b_l1, W_r1, W_l2, b_l2, W_r2):
    n, _ = x.shape
    d_out = W_l2.shape[1]
    outT = pl.pallas_call(
        _sage_fused,
        out_shape=jax.ShapeDtypeStruct((d_out, n), jnp.float32),
    )(adj, x.T,
      W_l0.T, b_l0.reshape(-1, 1), W_r0.T,
      W_l1.T, b_l1.reshape(-1, 1), W_r1.T,
      W_l2.T, b_l2.reshape(-1, 1), W_r2.T)
    return outT.T


# speedup vs baseline: 2374.1991x; 1.6713x over previous
import jax
import jax.numpy as jnp
from jax.experimental import pallas as pl

_PREC = jax.lax.Precision.DEFAULT


def _sage_fused(adj_ref, gx_ref,
                wl0_ref, bl0_ref, wr0_ref,
                wl1_ref, bl1_ref, wr1_ref,
                wl2_ref, bl2_ref, wr2_ref,
                out_ref):
    adj = adj_ref[...]
    g = gx_ref[...]

    deg = jnp.sum(adj, axis=0, keepdims=True)
    dinv = 1.0 / jnp.maximum(deg, 1.0)

    def layer(gh, wlT_ref, blc_ref, wrT_ref):
        aggT = jnp.dot(gh, adj, preferred_element_type=jnp.float32,
                       precision=_PREC)
        meanT = aggT * dinv
        lin_l = jnp.dot(wlT_ref[...], meanT, preferred_element_type=jnp.float32,
                        precision=_PREC)
        lin_r = jnp.dot(wrT_ref[...], gh, preferred_element_type=jnp.float32,
                        precision=_PREC)
        return lin_l + lin_r + blc_ref[...]

    g = jnp.maximum(layer(g, wl0_ref, bl0_ref, wr0_ref), 0.0)
    g = jnp.maximum(layer(g, wl1_ref, bl1_ref, wr1_ref), 0.0)
    out_ref[...] = layer(g, wl2_ref, bl2_ref, wr2_ref)


def kernel(x, adj, W_l0, b_l0, W_r0, W_l1, b_l1, W_r1, W_l2, b_l2, W_r2):
    n, _ = x.shape
    d_out = W_l2.shape[1]
    outT = pl.pallas_call(
        _sage_fused,
        out_shape=jax.ShapeDtypeStruct((d_out, n), jnp.float32),
    )(adj, x.T,
      W_l0.T, b_l0.reshape(-1, 1), W_r0.T,
      W_l1.T, b_l1.reshape(-1, 1), W_r1.T,
      W_l2.T, b_l2.reshape(-1, 1), W_r2.T)
    return outT.T
